# baseline (speedup 1.0000x reference)
import functools

import jax
import jax.numpy as jnp
from jax import lax
from jax.experimental import pallas as pl
from jax.experimental.pallas import tpu as pltpu
from jax.experimental.pallas import tpu_sc as plsc

B = 16
T = 512
D = 256
M = 4096
L = 16
NC, NS = 2, 16
NW = NC * NS
HALF = M // (NW // B)
CHUNK = 128
NCH = HALF // CHUNK
SRC = 96
XPR = B * T
SENT = B * T


@functools.partial(
    pl.kernel,
    out_type=(
        jax.ShapeDtypeStruct((B, M, D), jnp.float32),
        jax.ShapeDtypeStruct((B,), jnp.int32),
    ),
    mesh=plsc.VectorSubcoreMesh(core_axis_name="c", subcore_axis_name="s"),
    compiler_params=pltpu.CompilerParams(needs_layout_passes=False),
    scratch_types=[
        pltpu.VMEM((T,), jnp.int32),
        pltpu.VMEM((HALF,), jnp.int32),
        pltpu.VMEM((SRC + 8, D), jnp.float32),
        pltpu.VMEM((SRC + 8, D), jnp.float32),
        pltpu.VMEM((CHUNK, D), jnp.float32),
        pltpu.VMEM((CHUNK, D), jnp.float32),
        pltpu.VMEM((B,), jnp.int32),
        pltpu.SemaphoreType.DMA,
        pltpu.SemaphoreType.DMA,
        pltpu.SemaphoreType.DMA,
        pltpu.SemaphoreType.DMA,
        pltpu.SemaphoreType.DMA,
    ],
)
def _length_regulate(xp_hbm, dur_hbm, out_hbm, len_hbm,
                     dur_v, idx_v, srcA_v, srcB_v, outA_v, outB_v, len_v,
                     rsemA, rsemB, wsemA, wsemB, gsem):
    c = lax.axis_index("c")
    s = lax.axis_index("s")
    wid = s * NC + c
    b = s
    half = (s + c) % 2
    lo = half * HALF

    dbase = pl.multiple_of(b * T, T)
    pltpu.sync_copy(dur_hbm.at[pl.ds(dbase, T)], dur_v)

    zvec = jnp.full((L,), SENT, jnp.int32)

    def init_body(j, carry):
        idx_v[pl.ds(j * L, L)] = zvec
        return carry

    lax.fori_loop(0, HALF // L, init_body, 0)

    lanes = lax.iota(jnp.int32, L)

    def chunk_body(i, carry):
        d = dur_v[pl.ds(i * L, L)]
        c = plsc.cumsum(d) + carry
        start = c - d
        val = lanes + (i * L + b * T)
        for k in range(7):
            pos = start + k - lo
            msk = (d > k) & (pos >= 0) & (pos < HALF)
            pos_safe = jnp.clip(pos, 0, HALF - 1)
            plsc.store_scatter(idx_v, [pos_safe], val, mask=msk)
        return carry + jnp.sum(d)

    lax.fori_loop(0, T // L, chunk_body, jnp.int32(0))

    srcs = (srcA_v, srcB_v)
    outs = (outA_v, outB_v)
    rsems = (rsemA, rsemB)
    wsems = (wsemA, wsemB)
    bcap = b * T + T - 1

    zf = jnp.zeros((L,), jnp.float32)
    for k in range(D // L):
        srcA_v[SRC, pl.ds(k * L, L)] = zf
        srcB_v[SRC, pl.ds(k * L, L)] = zf

    def chunk_t0(j):
        lp = pl.multiple_of(j * CHUNK, CHUNK)
        return idx_v[pl.ds(lp, L)][0]

    def chunk_rb(j):
        return pl.multiple_of(jnp.minimum(chunk_t0(j) & -8, XPR - SRC), 8)

    def chunk_needs_read(j):
        return chunk_t0(j) != SENT

    def chunk_span_ok(j):
        lp = pl.multiple_of(j * CHUNK, CHUNK)
        tL = idx_v[pl.ds(lp + CHUNK - L, L)][L - 1]
        return (jnp.minimum(tL, bcap) - chunk_t0(j)) < (SRC - 8)

    def read_desc(j, slot):
        return pltpu.make_async_copy(
            xp_hbm.at[pl.ds(chunk_rb(j), SRC)],
            srcs[slot].at[pl.ds(0, SRC)], rsems[slot])

    def write_desc(j, slot):
        obase = pl.multiple_of(lo + j * CHUNK, CHUNK)
        return pltpu.make_async_copy(
            outs[slot], out_hbm.at[b, pl.ds(obase, CHUNK)], wsems[slot])

    def expand(j, slot):
        lp = pl.multiple_of(j * CHUNK, CHUNK)
        rb = chunk_rb(j)

        @plsc.parallel_loop(0, CHUNK // L)
        def _(g):
            raw = idx_v[pl.ds(lp + g * L, L)]
            rvec = jnp.where(raw == SENT, SRC,
                             jnp.minimum(raw - rb, SRC - 1))
            base = g * L

            def load_frame(r):
                return [srcs[slot][r, pl.ds(k * L, L)]
                        for k in range(D // L)]

            vals = load_frame(rvec[0])
            for l in range(1, L + 1):
                nxt = load_frame(rvec[l]) if l < L else None
                for k in range(D // L):
                    outs[slot][base + l - 1, pl.ds(k * L, L)] = vals[k]
                vals = nxt

    @pl.when(chunk_needs_read(0))
    def _():
        read_desc(0, 0).start()

    def pair_body(pair, carry):
        for slot in range(2):
            j = pair * 2 + slot

            @pl.when(j >= 2)
            def _():
                write_desc(j - 2, slot).wait()

            @pl.when((j + 1 < NCH) & chunk_needs_read(j + 1))
            def _():
                read_desc(j + 1, 1 - slot).start()

            @pl.when(chunk_needs_read(j))
            def _():
                read_desc(j, slot).wait()

            expand(j, slot)

            @pl.when(~chunk_span_ok(j))
            def _():
                lp = pl.multiple_of(j * CHUNK, CHUNK)

                def fb_body(g, carry):
                    raw = idx_v[pl.ds(lp + g * L, L)]
                    base = g * L
                    for l in range(L):
                        t = raw[l]

                        @pl.when(t < SENT)
                        def _():
                            rb8 = pl.multiple_of(
                                jnp.minimum(t & -8, XPR - 8), 8)
                            pltpu.async_copy(
                                xp_hbm.at[pl.ds(rb8, 8)],
                                srcs[slot].at[pl.ds(0, 8)], gsem).wait()
                            r8 = t - rb8
                            for k in range(D // L):
                                outs[slot][base + l, pl.ds(k * L, L)] = (
                                    srcs[slot][r8, pl.ds(k * L, L)])

                        @pl.when(t >= SENT)
                        def _():
                            for k in range(D // L):
                                outs[slot][base + l, pl.ds(k * L, L)] = (
                                    srcs[slot][SRC, pl.ds(k * L, L)])
                    return carry

                lax.fori_loop(0, CHUNK // L, fb_body, 0)

            write_desc(j, slot).start()
        return carry

    lax.fori_loop(0, NCH // 2, pair_body, 0)
    write_desc(NCH - 2, 0).wait()
    write_desc(NCH - 1, 1).wait()

    @pl.when(wid == 0)
    def _():
        lens = jnp.zeros((L,), jnp.int32)
        for bb in range(B):
            pltpu.sync_copy(dur_hbm.at[pl.ds(bb * T, T)], dur_v)

            def sum_body(i, acc):
                return acc + jnp.sum(dur_v[pl.ds(i * L, L)])

            s = lax.fori_loop(0, T // L, sum_body, jnp.int32(0))
            lens = jnp.where(lanes == bb, s, lens)
        len_v[...] = lens
        pltpu.sync_copy(len_v, len_hbm)


def kernel(x, duration, mel_max_len):
    del mel_max_len
    xp = x.reshape(B * T, D)
    dur = duration.astype(jnp.int32).reshape(B * T)
    out, mel_len = _length_regulate(xp, dur)
    return out, mel_len.astype(jnp.int64)


# speedup vs baseline: 1.0074x; 1.0074x over previous
import functools

import jax
import jax.numpy as jnp
from jax import lax
from jax.experimental import pallas as pl
from jax.experimental.pallas import tpu as pltpu
from jax.experimental.pallas import tpu_sc as plsc

B = 16
T = 512
D = 256
M = 4096
L = 16
NC, NS = 2, 16
NW = NC * NS
HALF = M // (NW // B)
CHUNK = 128
NCH = HALF // CHUNK
SRC = 96
XPR = B * T
SENT = B * T


@functools.partial(
    pl.kernel,
    out_type=(
        jax.ShapeDtypeStruct((B, M, D), jnp.float32),
        jax.ShapeDtypeStruct((B,), jnp.int32),
    ),
    mesh=plsc.VectorSubcoreMesh(core_axis_name="c", subcore_axis_name="s"),
    compiler_params=pltpu.CompilerParams(needs_layout_passes=False),
    scratch_types=[
        pltpu.VMEM((T,), jnp.int32),
        pltpu.VMEM((HALF,), jnp.int32),
        pltpu.VMEM((SRC + 8, D), jnp.float32),
        pltpu.VMEM((SRC + 8, D), jnp.float32),
        pltpu.VMEM((CHUNK, D), jnp.float32),
        pltpu.VMEM((CHUNK, D), jnp.float32),
        pltpu.VMEM((B,), jnp.int32),
        pltpu.SemaphoreType.DMA,
        pltpu.SemaphoreType.DMA,
        pltpu.SemaphoreType.DMA,
        pltpu.SemaphoreType.DMA,
        pltpu.SemaphoreType.DMA,
    ],
)
def _length_regulate(xp_hbm, dur_hbm, out_hbm, len_hbm,
                     dur_v, idx_v, srcA_v, srcB_v, outA_v, outB_v, len_v,
                     rsemA, rsemB, wsemA, wsemB, gsem):
    wid = lax.axis_index("s") * NC + lax.axis_index("c")
    b = wid // (NW // B)
    half = wid % (NW // B)
    lo = half * HALF

    dbase = pl.multiple_of(b * T, T)
    pltpu.sync_copy(dur_hbm.at[pl.ds(dbase, T)], dur_v)

    zvec = jnp.full((L,), SENT, jnp.int32)

    def init_body(j, carry):
        idx_v[pl.ds(j * L, L)] = zvec
        return carry

    lax.fori_loop(0, HALF // L, init_body, 0)

    lanes = lax.iota(jnp.int32, L)

    def chunk_body(i, carry):
        d = dur_v[pl.ds(i * L, L)]
        c = plsc.cumsum(d) + carry
        start = c - d
        val = lanes + (i * L + b * T)
        for k in range(7):
            pos = start + k - lo
            msk = (d > k) & (pos >= 0) & (pos < HALF)
            pos_safe = jnp.clip(pos, 0, HALF - 1)
            plsc.store_scatter(idx_v, [pos_safe], val, mask=msk)
        return carry + jnp.sum(d)

    lax.fori_loop(0, T // L, chunk_body, jnp.int32(0))

    srcs = (srcA_v, srcB_v)
    outs = (outA_v, outB_v)
    rsems = (rsemA, rsemB)
    wsems = (wsemA, wsemB)
    bcap = b * T + T - 1

    zf = jnp.zeros((L,), jnp.float32)
    for k in range(D // L):
        srcA_v[SRC, pl.ds(k * L, L)] = zf
        srcB_v[SRC, pl.ds(k * L, L)] = zf

    def chunk_t0(j):
        lp = pl.multiple_of(j * CHUNK, CHUNK)
        return idx_v[pl.ds(lp, L)][0]

    def chunk_rb(j):
        return pl.multiple_of(jnp.minimum(chunk_t0(j) & -8, XPR - SRC), 8)

    def chunk_needs_read(j):
        return chunk_t0(j) != SENT

    def chunk_span_ok(j):
        lp = pl.multiple_of(j * CHUNK, CHUNK)
        tL = idx_v[pl.ds(lp + CHUNK - L, L)][L - 1]
        return (jnp.minimum(tL, bcap) - chunk_t0(j)) < (SRC - 8)

    def read_desc(j, slot):
        return pltpu.make_async_copy(
            xp_hbm.at[pl.ds(chunk_rb(j), SRC)],
            srcs[slot].at[pl.ds(0, SRC)], rsems[slot])

    def write_desc(j, slot):
        obase = pl.multiple_of(lo + j * CHUNK, CHUNK)
        return pltpu.make_async_copy(
            outs[slot], out_hbm.at[b, pl.ds(obase, CHUNK)], wsems[slot])

    def expand(j, slot):
        lp = pl.multiple_of(j * CHUNK, CHUNK)
        rb = chunk_rb(j)

        @plsc.parallel_loop(0, CHUNK // L)
        def _(g):
            raw = idx_v[pl.ds(lp + g * L, L)]
            rvec = jnp.where(raw == SENT, SRC,
                             jnp.minimum(raw - rb, SRC - 1))
            base = g * L

            def load_frame(r):
                return [srcs[slot][r, pl.ds(k * L, L)]
                        for k in range(D // L)]

            vals = load_frame(rvec[0])
            for l in range(1, L + 1):
                nxt = load_frame(rvec[l]) if l < L else None
                for k in range(D // L):
                    outs[slot][base + l - 1, pl.ds(k * L, L)] = vals[k]
                vals = nxt

    @pl.when(chunk_needs_read(0))
    def _():
        read_desc(0, 0).start()

    def pair_body(pair, carry):
        for slot in range(2):
            j = pair * 2 + slot

            @pl.when(j >= 2)
            def _():
                write_desc(j - 2, slot).wait()

            @pl.when((j + 1 < NCH) & chunk_needs_read(j + 1))
            def _():
                read_desc(j + 1, 1 - slot).start()

            @pl.when(chunk_needs_read(j))
            def _():
                read_desc(j, slot).wait()

            expand(j, slot)

            @pl.when(~chunk_span_ok(j))
            def _():
                lp = pl.multiple_of(j * CHUNK, CHUNK)

                def fb_body(g, carry):
                    raw = idx_v[pl.ds(lp + g * L, L)]
                    base = g * L
                    for l in range(L):
                        t = raw[l]

                        @pl.when(t < SENT)
                        def _():
                            rb8 = pl.multiple_of(
                                jnp.minimum(t & -8, XPR - 8), 8)
                            pltpu.async_copy(
                                xp_hbm.at[pl.ds(rb8, 8)],
                                srcs[slot].at[pl.ds(0, 8)], gsem).wait()
                            r8 = t - rb8
                            for k in range(D // L):
                                outs[slot][base + l, pl.ds(k * L, L)] = (
                                    srcs[slot][r8, pl.ds(k * L, L)])

                        @pl.when(t >= SENT)
                        def _():
                            for k in range(D // L):
                                outs[slot][base + l, pl.ds(k * L, L)] = (
                                    srcs[slot][SRC, pl.ds(k * L, L)])
                    return carry

                lax.fori_loop(0, CHUNK // L, fb_body, 0)

            write_desc(j, slot).start()
        return carry

    lax.fori_loop(0, NCH // 2, pair_body, 0)
    write_desc(NCH - 2, 0).wait()
    write_desc(NCH - 1, 1).wait()

    @pl.when(wid == 0)
    def _():
        lens = jnp.zeros((L,), jnp.int32)
        for bb in range(B):
            pltpu.sync_copy(dur_hbm.at[pl.ds(bb * T, T)], dur_v)

            def sum_body(i, acc):
                return acc + jnp.sum(dur_v[pl.ds(i * L, L)])

            s = lax.fori_loop(0, T // L, sum_body, jnp.int32(0))
            lens = jnp.where(lanes == bb, s, lens)
        len_v[...] = lens
        pltpu.sync_copy(len_v, len_hbm)


def kernel(x, duration, mel_max_len):
    del mel_max_len
    xp = x.reshape(B * T, D)
    dur = duration.astype(jnp.int32).reshape(B * T)
    out, mel_len = _length_regulate(xp, dur)
    return out, mel_len.astype(jnp.int64)


# speedup vs baseline: 1.1809x; 1.1723x over previous
import functools

import jax
import jax.numpy as jnp
from jax import lax
from jax.experimental import pallas as pl
from jax.experimental.pallas import tpu as pltpu
from jax.experimental.pallas import tpu_sc as plsc

B = 16
T = 512
D = 256
M = 4096
L = 16
NC, NS = 2, 16
NW = NC * NS
HALF = M // (NW // B)
CHUNK = 128
NCH = HALF // CHUNK
SRC = 96
XPR = B * T
SENT = B * T


@functools.partial(
    pl.kernel,
    out_type=(
        jax.ShapeDtypeStruct((B, M, D), jnp.float32),
        jax.ShapeDtypeStruct((B, L), jnp.int32),
    ),
    mesh=plsc.VectorSubcoreMesh(core_axis_name="c", subcore_axis_name="s"),
    compiler_params=pltpu.CompilerParams(needs_layout_passes=False),
    scratch_types=[
        pltpu.VMEM((T,), jnp.int32),
        pltpu.VMEM((HALF,), jnp.int32),
        pltpu.VMEM((SRC + 8, D), jnp.float32),
        pltpu.VMEM((SRC + 8, D), jnp.float32),
        pltpu.VMEM((CHUNK, D), jnp.float32),
        pltpu.VMEM((CHUNK, D), jnp.float32),
        pltpu.VMEM((B,), jnp.int32),
        pltpu.SemaphoreType.DMA,
        pltpu.SemaphoreType.DMA,
        pltpu.SemaphoreType.DMA,
        pltpu.SemaphoreType.DMA,
        pltpu.SemaphoreType.DMA,
    ],
)
def _length_regulate(xp_hbm, dur_hbm, out_hbm, len_hbm,
                     dur_v, idx_v, srcA_v, srcB_v, outA_v, outB_v, len_v,
                     rsemA, rsemB, wsemA, wsemB, gsem):
    wid = lax.axis_index("s") * NC + lax.axis_index("c")
    b = wid // (NW // B)
    half = wid % (NW // B)
    lo = half * HALF

    dbase = pl.multiple_of(b * T, T)
    pltpu.sync_copy(dur_hbm.at[pl.ds(dbase, T)], dur_v)

    zvec = jnp.full((L,), SENT, jnp.int32)

    def init_body(j, carry):
        idx_v[pl.ds(j * L, L)] = zvec
        return carry

    lax.fori_loop(0, HALF // L, init_body, 0)

    lanes = lax.iota(jnp.int32, L)

    def chunk_body(i, carry):
        d = dur_v[pl.ds(i * L, L)]
        c = plsc.cumsum(d) + carry
        start = c - d
        val = lanes + (i * L + b * T)
        for k in range(7):
            pos = start + k - lo
            msk = (d > k) & (pos >= 0) & (pos < HALF)
            pos_safe = jnp.clip(pos, 0, HALF - 1)
            plsc.store_scatter(idx_v, [pos_safe], val, mask=msk)
        return carry + jnp.sum(d)

    total = lax.fori_loop(0, T // L, chunk_body, jnp.int32(0))

    @pl.when(half == 0)
    def _():
        len_v[...] = total + jnp.zeros((L,), jnp.int32)
        pltpu.sync_copy(len_v, len_hbm.at[b])

    srcs = (srcA_v, srcB_v)
    outs = (outA_v, outB_v)
    rsems = (rsemA, rsemB)
    wsems = (wsemA, wsemB)
    bcap = b * T + T - 1

    zf = jnp.zeros((L,), jnp.float32)
    for k in range(D // L):
        srcA_v[SRC, pl.ds(k * L, L)] = zf
        srcB_v[SRC, pl.ds(k * L, L)] = zf

    def chunk_t0(j):
        lp = pl.multiple_of(j * CHUNK, CHUNK)
        return idx_v[pl.ds(lp, L)][0]

    def chunk_rb(j):
        return pl.multiple_of(jnp.minimum(chunk_t0(j) & -8, XPR - SRC), 8)

    def chunk_needs_read(j):
        return chunk_t0(j) != SENT

    def chunk_span_ok(j):
        lp = pl.multiple_of(j * CHUNK, CHUNK)
        tL = idx_v[pl.ds(lp + CHUNK - L, L)][L - 1]
        return (jnp.minimum(tL, bcap) - chunk_t0(j)) < (SRC - 8)

    def read_desc(j, slot):
        return pltpu.make_async_copy(
            xp_hbm.at[pl.ds(chunk_rb(j), SRC)],
            srcs[slot].at[pl.ds(0, SRC)], rsems[slot])

    def write_desc(j, slot):
        obase = pl.multiple_of(lo + j * CHUNK, CHUNK)
        return pltpu.make_async_copy(
            outs[slot], out_hbm.at[b, pl.ds(obase, CHUNK)], wsems[slot])

    def expand(j, slot):
        lp = pl.multiple_of(j * CHUNK, CHUNK)
        rb = chunk_rb(j)

        @plsc.parallel_loop(0, CHUNK // L)
        def _(g):
            raw = idx_v[pl.ds(lp + g * L, L)]
            rvec = jnp.where(raw == SENT, SRC,
                             jnp.minimum(raw - rb, SRC - 1))
            base = g * L

            def load_frame(r):
                return [srcs[slot][r, pl.ds(k * L, L)]
                        for k in range(D // L)]

            vals = load_frame(rvec[0])
            for l in range(1, L + 1):
                nxt = load_frame(rvec[l]) if l < L else None
                for k in range(D // L):
                    outs[slot][base + l - 1, pl.ds(k * L, L)] = vals[k]
                vals = nxt

    @pl.when(chunk_needs_read(0))
    def _():
        read_desc(0, 0).start()

    def pair_body(pair, carry):
        for slot in range(2):
            j = pair * 2 + slot

            @pl.when(j >= 2)
            def _():
                write_desc(j - 2, slot).wait()

            @pl.when((j + 1 < NCH) & chunk_needs_read(j + 1))
            def _():
                read_desc(j + 1, 1 - slot).start()

            @pl.when(chunk_needs_read(j))
            def _():
                read_desc(j, slot).wait()

            expand(j, slot)

            @pl.when(~chunk_span_ok(j))
            def _():
                lp = pl.multiple_of(j * CHUNK, CHUNK)

                def fb_body(g, carry):
                    raw = idx_v[pl.ds(lp + g * L, L)]
                    base = g * L
                    for l in range(L):
                        t = raw[l]
                        rb8 = pl.multiple_of(jnp.minimum(t & -8, XPR - 8), 8)
                        pltpu.async_copy(
                            xp_hbm.at[pl.ds(rb8, 8)],
                            srcs[slot].at[pl.ds(0, 8)], gsem).wait()
                        r8 = jnp.where(t == SENT, SRC, t - rb8)
                        for k in range(D // L):
                            outs[slot][base + l, pl.ds(k * L, L)] = (
                                srcs[slot][r8, pl.ds(k * L, L)])
                    return carry

                lax.fori_loop(0, CHUNK // L, fb_body, 0)

            write_desc(j, slot).start()
        return carry

    lax.fori_loop(0, NCH // 2, pair_body, 0)
    write_desc(NCH - 2, 0).wait()
    write_desc(NCH - 1, 1).wait()


def kernel(x, duration, mel_max_len):
    del mel_max_len
    xp = x.reshape(B * T, D)
    dur = duration.astype(jnp.int32).reshape(B * T)
    out, len_rows = _length_regulate(xp, dur)
    return out, len_rows[:, 0].astype(jnp.int64)


# speedup vs baseline: 1.1815x; 1.0005x over previous
import functools

import jax
import jax.numpy as jnp
from jax import lax
from jax.experimental import pallas as pl
from jax.experimental.pallas import tpu as pltpu
from jax.experimental.pallas import tpu_sc as plsc

B = 16
T = 512
D = 256
M = 4096
L = 16
NC, NS = 2, 16
NW = NC * NS
HALF = M // (NW // B)
CHUNK = 128
NCH = HALF // CHUNK
SRC = 96
XPR = B * T
SENT = B * T


@functools.partial(
    pl.kernel,
    out_type=(
        jax.ShapeDtypeStruct((B, M, D), jnp.float32),
        jax.ShapeDtypeStruct((B, L), jnp.int32),
    ),
    mesh=plsc.VectorSubcoreMesh(core_axis_name="c", subcore_axis_name="s"),
    compiler_params=pltpu.CompilerParams(needs_layout_passes=False),
    scratch_types=[
        pltpu.VMEM((T,), jnp.int32),
        pltpu.VMEM((HALF,), jnp.int32),
        pltpu.VMEM((SRC + 8, D), jnp.float32),
        pltpu.VMEM((SRC + 8, D), jnp.float32),
        pltpu.VMEM((CHUNK, D), jnp.float32),
        pltpu.VMEM((CHUNK, D), jnp.float32),
        pltpu.VMEM((B,), jnp.int32),
        pltpu.SemaphoreType.DMA,
        pltpu.SemaphoreType.DMA,
        pltpu.SemaphoreType.DMA,
        pltpu.SemaphoreType.DMA,
        pltpu.SemaphoreType.DMA,
    ],
)
def _length_regulate(xp_hbm, dur_hbm, out_hbm, len_hbm,
                     dur_v, idx_v, srcA_v, srcB_v, outA_v, outB_v, len_v,
                     rsemA, rsemB, wsemA, wsemB, gsem):
    c = lax.axis_index("c")
    s_idx = lax.axis_index("s")
    wid = s_idx * NC + c
    b = s_idx
    half = (s_idx + c) % 2
    lo = half * HALF

    dbase = pl.multiple_of(b * T, T)
    pltpu.sync_copy(dur_hbm.at[pl.ds(dbase, T)], dur_v)

    zvec = jnp.full((L,), SENT, jnp.int32)

    def init_body(j, carry):
        idx_v[pl.ds(j * L, L)] = zvec
        return carry

    lax.fori_loop(0, HALF // L, init_body, 0)

    lanes = lax.iota(jnp.int32, L)

    def chunk_body(i, carry):
        d = dur_v[pl.ds(i * L, L)]
        c = plsc.cumsum(d) + carry
        start = c - d
        val = lanes + (i * L + b * T)
        for k in range(7):
            pos = start + k - lo
            msk = (d > k) & (pos >= 0) & (pos < HALF)
            pos_safe = jnp.clip(pos, 0, HALF - 1)
            plsc.store_scatter(idx_v, [pos_safe], val, mask=msk)
        return carry + jnp.sum(d)

    total = lax.fori_loop(0, T // L, chunk_body, jnp.int32(0))

    @pl.when(half == 0)
    def _():
        len_v[...] = total + jnp.zeros((L,), jnp.int32)
        pltpu.sync_copy(len_v, len_hbm.at[b])

    srcs = (srcA_v, srcB_v)
    outs = (outA_v, outB_v)
    rsems = (rsemA, rsemB)
    wsems = (wsemA, wsemB)
    bcap = b * T + T - 1

    zf = jnp.zeros((L,), jnp.float32)
    for k in range(D // L):
        srcA_v[SRC, pl.ds(k * L, L)] = zf
        srcB_v[SRC, pl.ds(k * L, L)] = zf

    def chunk_t0(j):
        lp = pl.multiple_of(j * CHUNK, CHUNK)
        return idx_v[pl.ds(lp, L)][0]

    def chunk_rb(j):
        return pl.multiple_of(jnp.minimum(chunk_t0(j) & -8, XPR - SRC), 8)

    def chunk_needs_read(j):
        return chunk_t0(j) != SENT

    def chunk_span_ok(j):
        lp = pl.multiple_of(j * CHUNK, CHUNK)
        tL = idx_v[pl.ds(lp + CHUNK - L, L)][L - 1]
        return (jnp.minimum(tL, bcap) - chunk_t0(j)) < (SRC - 8)

    def read_desc(j, slot):
        return pltpu.make_async_copy(
            xp_hbm.at[pl.ds(chunk_rb(j), SRC)],
            srcs[slot].at[pl.ds(0, SRC)], rsems[slot])

    def write_desc(j, slot):
        obase = pl.multiple_of(lo + j * CHUNK, CHUNK)
        return pltpu.make_async_copy(
            outs[slot], out_hbm.at[b, pl.ds(obase, CHUNK)], wsems[slot])

    def expand(j, slot):
        lp = pl.multiple_of(j * CHUNK, CHUNK)
        rb = chunk_rb(j)

        @plsc.parallel_loop(0, CHUNK // L)
        def _(g):
            raw = idx_v[pl.ds(lp + g * L, L)]
            rvec = jnp.where(raw == SENT, SRC,
                             jnp.minimum(raw - rb, SRC - 1))
            base = g * L

            def load_frame(r):
                return [srcs[slot][r, pl.ds(k * L, L)]
                        for k in range(D // L)]

            vals = load_frame(rvec[0])
            for l in range(1, L + 1):
                nxt = load_frame(rvec[l]) if l < L else None
                for k in range(D // L):
                    outs[slot][base + l - 1, pl.ds(k * L, L)] = vals[k]
                vals = nxt

    @pl.when(chunk_needs_read(0))
    def _():
        read_desc(0, 0).start()

    def pair_body(pair, carry):
        for slot in range(2):
            j = pair * 2 + slot

            @pl.when(j >= 2)
            def _():
                write_desc(j - 2, slot).wait()

            @pl.when((j + 1 < NCH) & chunk_needs_read(j + 1))
            def _():
                read_desc(j + 1, 1 - slot).start()

            @pl.when(chunk_needs_read(j))
            def _():
                read_desc(j, slot).wait()

            expand(j, slot)

            @pl.when(~chunk_span_ok(j))
            def _():
                lp = pl.multiple_of(j * CHUNK, CHUNK)

                def fb_body(g, carry):
                    raw = idx_v[pl.ds(lp + g * L, L)]
                    base = g * L
                    for l in range(L):
                        t = raw[l]
                        rb8 = pl.multiple_of(jnp.minimum(t & -8, XPR - 8), 8)
                        pltpu.async_copy(
                            xp_hbm.at[pl.ds(rb8, 8)],
                            srcs[slot].at[pl.ds(0, 8)], gsem).wait()
                        r8 = jnp.where(t == SENT, SRC, t - rb8)
                        for k in range(D // L):
                            outs[slot][base + l, pl.ds(k * L, L)] = (
                                srcs[slot][r8, pl.ds(k * L, L)])
                    return carry

                lax.fori_loop(0, CHUNK // L, fb_body, 0)

            write_desc(j, slot).start()
        return carry

    lax.fori_loop(0, NCH // 2, pair_body, 0)
    write_desc(NCH - 2, 0).wait()
    write_desc(NCH - 1, 1).wait()


def kernel(x, duration, mel_max_len):
    del mel_max_len
    xp = x.reshape(B * T, D)
    dur = duration.astype(jnp.int32).reshape(B * T)
    out, len_rows = _length_regulate(xp, dur)
    return out, len_rows[:, 0].astype(jnp.int64)


# speedup vs baseline: 1.1836x; 1.0017x over previous
import functools

import jax
import jax.numpy as jnp
from jax import lax
from jax.experimental import pallas as pl
from jax.experimental.pallas import tpu as pltpu
from jax.experimental.pallas import tpu_sc as plsc

B = 16
T = 512
D = 256
M = 4096
L = 16
NC, NS = 2, 16
NW = NC * NS
HALF = M // (NW // B)
CHUNK = 128
NCH = HALF // CHUNK
SRC = 96
XPR = B * T
SENT = B * T


@functools.partial(
    pl.kernel,
    out_type=(
        jax.ShapeDtypeStruct((B, M, D), jnp.float32),
        jax.ShapeDtypeStruct((B, L), jnp.int32),
    ),
    mesh=plsc.VectorSubcoreMesh(core_axis_name="c", subcore_axis_name="s"),
    compiler_params=pltpu.CompilerParams(needs_layout_passes=False),
    scratch_types=[
        pltpu.VMEM((T,), jnp.int32),
        pltpu.VMEM((HALF,), jnp.int32),
        pltpu.VMEM((SRC + 8, D), jnp.float32),
        pltpu.VMEM((SRC + 8, D), jnp.float32),
        pltpu.VMEM((CHUNK, D), jnp.float32),
        pltpu.VMEM((CHUNK, D), jnp.float32),
        pltpu.VMEM((B,), jnp.int32),
        pltpu.SemaphoreType.DMA,
        pltpu.SemaphoreType.DMA,
        pltpu.SemaphoreType.DMA,
        pltpu.SemaphoreType.DMA,
        pltpu.SemaphoreType.DMA,
    ],
)
def _length_regulate(xp_hbm, dur_hbm, out_hbm, len_hbm,
                     dur_v, idx_v, srcA_v, srcB_v, outA_v, outB_v, len_v,
                     rsemA, rsemB, wsemA, wsemB, gsem):
    c = lax.axis_index("c")
    s_idx = lax.axis_index("s")
    b = s_idx
    half = (s_idx + c) % 2
    lo = half * HALF

    dbase = pl.multiple_of(b * T, T)
    pltpu.sync_copy(dur_hbm.at[pl.ds(dbase, T)], dur_v)

    zvec = jnp.full((L,), SENT, jnp.int32)

    def init_body(j, carry):
        idx_v[pl.ds(j * L, L)] = zvec
        return carry

    lax.fori_loop(0, HALF // L, init_body, 0)

    lanes = lax.iota(jnp.int32, L)

    def chunk_body(i, carry):
        d = dur_v[pl.ds(i * L, L)]
        c = plsc.cumsum(d) + carry
        start = c - d
        val = lanes + (i * L + b * T)
        for k in range(7):
            pos = start + k - lo
            msk = (d > k) & (pos >= 0) & (pos < HALF)
            pos_safe = jnp.clip(pos, 0, HALF - 1)
            plsc.store_scatter(idx_v, [pos_safe], val, mask=msk)
        return carry + jnp.sum(d)

    total = lax.fori_loop(0, T // L, chunk_body, jnp.int32(0))

    @pl.when(half == 0)
    def _():
        len_v[...] = total + jnp.zeros((L,), jnp.int32)
        pltpu.sync_copy(len_v, len_hbm.at[b])

    srcs = (srcA_v, srcB_v)
    outs = (outA_v, outB_v)
    rsems = (rsemA, rsemB)
    wsems = (wsemA, wsemB)
    bcap = b * T + T - 1

    zf = jnp.zeros((L,), jnp.float32)
    for k in range(D // L):
        srcA_v[SRC, pl.ds(k * L, L)] = zf
        srcB_v[SRC, pl.ds(k * L, L)] = zf

    def chunk_t0(j):
        lp = pl.multiple_of(j * CHUNK, CHUNK)
        return idx_v[pl.ds(lp, L)][0]

    def chunk_rb(j):
        return pl.multiple_of(jnp.minimum(chunk_t0(j) & -8, XPR - SRC), 8)

    def chunk_needs_read(j):
        return chunk_t0(j) != SENT

    def chunk_span_ok(j):
        lp = pl.multiple_of(j * CHUNK, CHUNK)
        tL = idx_v[pl.ds(lp + CHUNK - L, L)][L - 1]
        return (jnp.minimum(tL, bcap) - chunk_t0(j)) < (SRC - 8)

    def read_desc(j, slot):
        return pltpu.make_async_copy(
            xp_hbm.at[pl.ds(chunk_rb(j), SRC)],
            srcs[slot].at[pl.ds(0, SRC)], rsems[slot])

    def write_desc(j, slot):
        obase = pl.multiple_of(lo + j * CHUNK, CHUNK)
        return pltpu.make_async_copy(
            outs[slot], out_hbm.at[b, pl.ds(obase, CHUNK)], wsems[slot])

    def expand(j, slot):
        lp = pl.multiple_of(j * CHUNK, CHUNK)
        rb = chunk_rb(j)

        @plsc.parallel_loop(0, CHUNK // L)
        def _(g):
            raw = idx_v[pl.ds(lp + g * L, L)]
            rvec = jnp.where(raw == SENT, SRC,
                             jnp.minimum(raw - rb, SRC - 1))
            base = g * L

            def load_frame(r):
                return [srcs[slot][r, pl.ds(k * L, L)]
                        for k in range(D // L)]

            vals = load_frame(rvec[0])
            for l in range(1, L + 1):
                nxt = load_frame(rvec[l]) if l < L else None
                for k in range(D // L):
                    outs[slot][base + l - 1, pl.ds(k * L, L)] = vals[k]
                vals = nxt

    @pl.when(chunk_needs_read(0))
    def _():
        read_desc(0, 0).start()

    def pair_body(pair, carry):
        for slot in range(2):
            j = pair * 2 + slot

            @pl.when(j >= 2)
            def _():
                write_desc(j - 2, slot).wait()

            @pl.when((j + 1 < NCH) & chunk_needs_read(j + 1))
            def _():
                read_desc(j + 1, 1 - slot).start()

            @pl.when(chunk_needs_read(j))
            def _():
                read_desc(j, slot).wait()

            expand(j, slot)

            @pl.when(~chunk_span_ok(j))
            def _():
                lp = pl.multiple_of(j * CHUNK, CHUNK)

                def fb_body(g, carry):
                    raw = idx_v[pl.ds(lp + g * L, L)]
                    base = g * L
                    for l in range(L):
                        t = raw[l]
                        rb8 = pl.multiple_of(jnp.minimum(t & -8, XPR - 8), 8)
                        pltpu.async_copy(
                            xp_hbm.at[pl.ds(rb8, 8)],
                            srcs[slot].at[pl.ds(0, 8)], gsem).wait()
                        r8 = jnp.where(t == SENT, SRC, t - rb8)
                        for k in range(D // L):
                            outs[slot][base + l, pl.ds(k * L, L)] = (
                                srcs[slot][r8, pl.ds(k * L, L)])
                    return carry

                lax.fori_loop(0, CHUNK // L, fb_body, 0)

            write_desc(j, slot).start()
        return carry

    lax.fori_loop(0, NCH // 2, pair_body, 0)
    write_desc(NCH - 2, 0).wait()
    write_desc(NCH - 1, 1).wait()


def kernel(x, duration, mel_max_len):
    del mel_max_len
    xp = x.reshape(B * T, D)
    dur = duration.astype(jnp.int32).reshape(B * T)
    out, len_rows = _length_regulate(xp, dur)
    return out, len_rows[:, 0].astype(jnp.int64)


# speedup vs baseline: 1.1893x; 1.0048x over previous
import functools

import jax
import jax.numpy as jnp
from jax import lax
from jax.experimental import pallas as pl
from jax.experimental.pallas import tpu as pltpu
from jax.experimental.pallas import tpu_sc as plsc

B = 16
T = 512
D = 256
M = 4096
L = 16
NC, NS = 2, 16
NW = NC * NS
HALF = M // (NW // B)
CHUNK = 128
NCH = HALF // CHUNK
SRC = 80
XPR = B * T
SENT = B * T


@functools.partial(
    pl.kernel,
    out_type=(
        jax.ShapeDtypeStruct((B, M, D), jnp.float32),
        jax.ShapeDtypeStruct((B, L), jnp.int32),
    ),
    mesh=plsc.VectorSubcoreMesh(core_axis_name="c", subcore_axis_name="s"),
    compiler_params=pltpu.CompilerParams(needs_layout_passes=False),
    scratch_types=[
        pltpu.VMEM((T,), jnp.int32),
        pltpu.VMEM((HALF,), jnp.int32),
        pltpu.VMEM((SRC + 8, D), jnp.float32),
        pltpu.VMEM((SRC + 8, D), jnp.float32),
        pltpu.VMEM((CHUNK, D), jnp.float32),
        pltpu.VMEM((CHUNK, D), jnp.float32),
        pltpu.VMEM((B,), jnp.int32),
        pltpu.SemaphoreType.DMA,
        pltpu.SemaphoreType.DMA,
        pltpu.SemaphoreType.DMA,
        pltpu.SemaphoreType.DMA,
        pltpu.SemaphoreType.DMA,
    ],
)
def _length_regulate(xp_hbm, dur_hbm, out_hbm, len_hbm,
                     dur_v, idx_v, srcA_v, srcB_v, outA_v, outB_v, len_v,
                     rsemA, rsemB, wsemA, wsemB, gsem):
    c = lax.axis_index("c")
    s_idx = lax.axis_index("s")
    b = s_idx
    half = (s_idx + c) % 2
    lo = half * HALF

    dbase = pl.multiple_of(b * T, T)
    pltpu.sync_copy(dur_hbm.at[pl.ds(dbase, T)], dur_v)

    zvec = jnp.full((L,), SENT, jnp.int32)

    def init_body(j, carry):
        idx_v[pl.ds(j * L, L)] = zvec
        return carry

    lax.fori_loop(0, HALF // L, init_body, 0)

    lanes = lax.iota(jnp.int32, L)

    def chunk_body(i, carry):
        d = dur_v[pl.ds(i * L, L)]
        c = plsc.cumsum(d) + carry
        start = c - d
        val = lanes + (i * L + b * T)
        for k in range(7):
            pos = start + k - lo
            msk = (d > k) & (pos >= 0) & (pos < HALF)
            pos_safe = jnp.clip(pos, 0, HALF - 1)
            plsc.store_scatter(idx_v, [pos_safe], val, mask=msk)
        return carry + jnp.sum(d)

    total = lax.fori_loop(0, T // L, chunk_body, jnp.int32(0))

    @pl.when(half == 0)
    def _():
        len_v[...] = total + jnp.zeros((L,), jnp.int32)
        pltpu.sync_copy(len_v, len_hbm.at[b])

    srcs = (srcA_v, srcB_v)
    outs = (outA_v, outB_v)
    rsems = (rsemA, rsemB)
    wsems = (wsemA, wsemB)
    bcap = b * T + T - 1

    zf = jnp.zeros((L,), jnp.float32)
    for k in range(D // L):
        srcA_v[SRC, pl.ds(k * L, L)] = zf
        srcB_v[SRC, pl.ds(k * L, L)] = zf

    def chunk_t0(j):
        lp = pl.multiple_of(j * CHUNK, CHUNK)
        return idx_v[pl.ds(lp, L)][0]

    def chunk_rb(j):
        return pl.multiple_of(jnp.minimum(chunk_t0(j) & -8, XPR - SRC), 8)

    def chunk_needs_read(j):
        return chunk_t0(j) != SENT

    def chunk_span_ok(j):
        lp = pl.multiple_of(j * CHUNK, CHUNK)
        tL = idx_v[pl.ds(lp + CHUNK - L, L)][L - 1]
        return (jnp.minimum(tL, bcap) - chunk_t0(j)) < (SRC - 8)

    def read_desc(j, slot):
        return pltpu.make_async_copy(
            xp_hbm.at[pl.ds(chunk_rb(j), SRC)],
            srcs[slot].at[pl.ds(0, SRC)], rsems[slot])

    def write_desc(j, slot):
        obase = pl.multiple_of(lo + j * CHUNK, CHUNK)
        return pltpu.make_async_copy(
            outs[slot], out_hbm.at[b, pl.ds(obase, CHUNK)], wsems[slot])

    def expand(j, slot):
        lp = pl.multiple_of(j * CHUNK, CHUNK)
        rb = chunk_rb(j)

        @plsc.parallel_loop(0, CHUNK // L)
        def _(g):
            raw = idx_v[pl.ds(lp + g * L, L)]
            rvec = jnp.where(raw == SENT, SRC,
                             jnp.minimum(raw - rb, SRC - 1))
            base = g * L

            def load_frame(r):
                return [srcs[slot][r, pl.ds(k * L, L)]
                        for k in range(D // L)]

            vals = load_frame(rvec[0])
            for l in range(1, L + 1):
                nxt = load_frame(rvec[l]) if l < L else None
                for k in range(D // L):
                    outs[slot][base + l - 1, pl.ds(k * L, L)] = vals[k]
                vals = nxt

    @pl.when(chunk_needs_read(0))
    def _():
        read_desc(0, 0).start()

    def pair_body(pair, carry):
        for slot in range(2):
            j = pair * 2 + slot

            @pl.when(j >= 2)
            def _():
                write_desc(j - 2, slot).wait()

            @pl.when((j + 1 < NCH) & chunk_needs_read(j + 1))
            def _():
                read_desc(j + 1, 1 - slot).start()

            @pl.when(chunk_needs_read(j))
            def _():
                read_desc(j, slot).wait()

            expand(j, slot)

            @pl.when(~chunk_span_ok(j))
            def _():
                lp = pl.multiple_of(j * CHUNK, CHUNK)

                def fb_body(g, carry):
                    raw = idx_v[pl.ds(lp + g * L, L)]
                    base = g * L
                    for l in range(L):
                        t = raw[l]
                        rb8 = pl.multiple_of(jnp.minimum(t & -8, XPR - 8), 8)
                        pltpu.async_copy(
                            xp_hbm.at[pl.ds(rb8, 8)],
                            srcs[slot].at[pl.ds(0, 8)], gsem).wait()
                        r8 = jnp.where(t == SENT, SRC, t - rb8)
                        for k in range(D // L):
                            outs[slot][base + l, pl.ds(k * L, L)] = (
                                srcs[slot][r8, pl.ds(k * L, L)])
                    return carry

                lax.fori_loop(0, CHUNK // L, fb_body, 0)

            write_desc(j, slot).start()
        return carry

    lax.fori_loop(0, NCH // 2, pair_body, 0)
    write_desc(NCH - 2, 0).wait()
    write_desc(NCH - 1, 1).wait()


def kernel(x, duration, mel_max_len):
    del mel_max_len
    xp = x.reshape(B * T, D)
    dur = duration.astype(jnp.int32).reshape(B * T)
    out, len_rows = _length_regulate(xp, dur)
    return out, len_rows[:, 0].astype(jnp.int64)
